# baseline (speedup 1.0000x reference)
import functools

import jax
import jax.numpy as jnp
from jax import lax
from jax.experimental import pallas as pl
from jax.experimental.pallas import tpu as pltpu
from jax.experimental.pallas import tpu_sc as plsc

_B = 16
_L = 4096
_D = 512
_TOTAL = _B * _L // 2
_NW = 32
_RPW = _B * _L // _NW
_CHUNK = 64
_NCHUNK = _RPW // _CHUNK
_LANES = 16


def _sc_body(flat_hbm, cu_hbm, out_hbm, mask_hbm,
             cu_v, idx0, idx1, buf0, buf1, zbuf, mbuf,
             isem0, isem1, osem0, osem1, zsem, msem):
    cid = lax.axis_index("c")
    sid = lax.axis_index("s")
    w = sid * 2 + cid
    b = w // 2
    half = (w % 2) ^ (b % 2)
    t0 = half * _RPW
    obase = (2 * b + half) * _RPW

    pltpu.sync_copy(cu_hbm, cu_v.at[pl.ds(0, _B + 1)])
    iota = lax.iota(jnp.int32, _LANES)
    start_b = cu_v[pl.ds(b, _LANES)][0]
    end_b = cu_v[pl.ds(b + 1, _LANES)][0]
    nv = jnp.clip(end_b - start_b - t0, 0, _RPW)
    s0 = start_b + t0
    pcv = (nv + _CHUNK - 1) // _CHUNK

    bufs = (buf0, buf1)
    idxs = (idx0, idx1)
    isems = (isem0, isem1)
    osems = (osem0, osem1)

    def start_in(c, bi):
        s = s0 + c * _CHUNK
        for kk in range(_CHUNK // _LANES):
            v = jnp.minimum(s + kk * _LANES + iota, _TOTAL - 1)
            idxs[bi][pl.ds(kk * _LANES, _LANES)] = v
        pltpu.make_async_copy(flat_hbm.at[idxs[bi]], bufs[bi], isems[bi]).start()

    for bi in range(2):
        @pl.when(bi < pcv)
        def _(bi=bi):
            start_in(bi, bi)

    zerosf = jnp.zeros((_LANES,), jnp.float32)

    def _zrow(row, _):
        for kk in range(_D // _LANES):
            zbuf[row, pl.ds(kk * _LANES, _LANES)] = zerosf
        return 0
    lax.fori_loop(0, _CHUNK, _zrow, 0)

    def _zfill(h, _):
        dst = pl.multiple_of(obase + (pcv + h) * _CHUNK, _CHUNK)
        pltpu.make_async_copy(zbuf, out_hbm.at[pl.ds(dst, _CHUNK)], zsem).start()
        return 0
    nzfires = _NCHUNK - pcv
    lax.fori_loop(0, nzfires, _zfill, 0)

    mt0 = (w % 2) * _RPW
    mnv = jnp.clip(end_b - start_b - mt0, 0, _RPW)
    for j in range(_RPW // _LANES):
        m = jnp.where(j * _LANES + iota < mnv, 1.0, 0.0).astype(jnp.float32)
        mbuf[pl.ds(j * _LANES, _LANES)] = m
    pltpu.make_async_copy(
        mbuf, mask_hbm.at[pl.ds(pl.multiple_of(w * _RPW, _RPW), _RPW)], msem
    ).start()

    def _pipe(g, _):
        for bi in range(2):
            c = g * 2 + bi

            @pl.when(c < pcv)
            def _(c=c, bi=bi):
                pltpu.make_async_copy(
                    flat_hbm.at[idxs[bi]], bufs[bi], isems[bi]
                ).wait()
                nvc = jnp.clip(nv - c * _CHUNK, 0, _CHUNK)

                def _ztail(row, _c):
                    for kk in range(_D // _LANES):
                        bufs[bi][row, pl.ds(kk * _LANES, _LANES)] = zerosf
                    return 0
                lax.fori_loop(nvc, _CHUNK, _ztail, 0)

                pltpu.make_async_copy(
                    bufs[bi],
                    out_hbm.at[pl.ds(
                        pl.multiple_of(obase + c * _CHUNK, _CHUNK), _CHUNK)],
                    osems[bi],
                ).start()

                @pl.when(c + 2 < pcv)
                def _():
                    pltpu.make_async_copy(
                        bufs[bi],
                        out_hbm.at[pl.ds(0, _CHUNK)],
                        osems[bi],
                    ).wait()
                    start_in(c + 2, bi)
        return 0
    lax.fori_loop(0, (pcv + 1) // 2, _pipe, 0)

    for bi in range(2):
        @pl.when(bi < pcv)
        def _(bi=bi):
            pltpu.make_async_copy(
                bufs[bi],
                out_hbm.at[pl.ds(0, _CHUNK)],
                osems[bi],
            ).wait()

    pltpu.make_async_copy(
        mbuf, mask_hbm.at[pl.ds(0, _RPW)], msem).wait()

    def _zdrain(h, _):
        pltpu.make_async_copy(zbuf, out_hbm.at[pl.ds(0, _CHUNK)], zsem).wait()
        return 0
    lax.fori_loop(0, nzfires, _zdrain, 0)


@jax.jit
def _padded_gather(flat, cu):
    mesh = plsc.VectorSubcoreMesh(core_axis_name="c", subcore_axis_name="s")
    return pl.kernel(
        _sc_body,
        out_type=(
            jax.ShapeDtypeStruct((_B * _L, _D), jnp.float32),
            jax.ShapeDtypeStruct((_B * _L,), jnp.float32),
        ),
        mesh=mesh,
        scratch_types=[
            pltpu.VMEM((3 * _LANES,), jnp.int32),
            pltpu.VMEM((_CHUNK,), jnp.int32),
            pltpu.VMEM((_CHUNK,), jnp.int32),
            pltpu.VMEM((_CHUNK, _D), jnp.float32),
            pltpu.VMEM((_CHUNK, _D), jnp.float32),
            pltpu.VMEM((_CHUNK, _D), jnp.float32),
            pltpu.VMEM((_RPW,), jnp.float32),
            pltpu.SemaphoreType.DMA,
            pltpu.SemaphoreType.DMA,
            pltpu.SemaphoreType.DMA,
            pltpu.SemaphoreType.DMA,
            pltpu.SemaphoreType.DMA,
            pltpu.SemaphoreType.DMA,
        ],
    )(flat, cu)


def kernel(flat, cu_seqlens):
    return _padded_gather(flat, cu_seqlens)


# speedup vs baseline: 1.0087x; 1.0087x over previous
import functools

import jax
import jax.numpy as jnp
from jax import lax
from jax.experimental import pallas as pl
from jax.experimental.pallas import tpu as pltpu
from jax.experimental.pallas import tpu_sc as plsc

_B = 16
_L = 4096
_D = 512
_TOTAL = _B * _L // 2
_NW = 32
_RPW = _B * _L // _NW
_CHUNK = 64
_NCHUNK = _RPW // _CHUNK
_ZROWS = 32
_LANES = 16


def _sc_body(flat_hbm, cu_hbm, out_hbm, mask_hbm,
             cu_v, idx0, idx1, idx2, buf0, buf1, buf2, zbuf, mbuf,
             isem0, isem1, isem2, osem0, osem1, osem2, zsem, msem):
    cid = lax.axis_index("c")
    sid = lax.axis_index("s")
    w = sid * 2 + cid
    b = w // 2
    half = (w % 2) ^ (b % 2)
    t0 = half * _RPW
    obase = (2 * b + half) * _RPW

    pltpu.sync_copy(cu_hbm, cu_v.at[pl.ds(0, _B + 1)])
    iota = lax.iota(jnp.int32, _LANES)
    start_b = cu_v[pl.ds(b, _LANES)][0]
    end_b = cu_v[pl.ds(b + 1, _LANES)][0]
    nv = jnp.clip(end_b - start_b - t0, 0, _RPW)
    s0 = start_b + t0
    pcv = (nv + _CHUNK - 1) // _CHUNK

    bufs = (buf0, buf1, buf2)
    idxs = (idx0, idx1, idx2)
    isems = (isem0, isem1, isem2)
    osems = (osem0, osem1, osem2)

    def start_in(c, bi):
        s = s0 + c * _CHUNK
        for kk in range(_CHUNK // _LANES):
            v = jnp.minimum(s + kk * _LANES + iota, _TOTAL - 1)
            idxs[bi][pl.ds(kk * _LANES, _LANES)] = v
        pltpu.make_async_copy(flat_hbm.at[idxs[bi]], bufs[bi], isems[bi]).start()

    for bi in range(3):
        @pl.when(bi < pcv)
        def _(bi=bi):
            start_in(bi, bi)

    zerosf = jnp.zeros((_LANES,), jnp.float32)

    def _zrow(row, _):
        for kk in range(_D // _LANES):
            zbuf[row, pl.ds(kk * _LANES, _LANES)] = zerosf
        return 0
    lax.fori_loop(0, _ZROWS, _zrow, 0)

    def _zfill(h, _):
        dst = pl.multiple_of(obase + pcv * _CHUNK + h * _ZROWS, _ZROWS)
        pltpu.make_async_copy(zbuf, out_hbm.at[pl.ds(dst, _ZROWS)], zsem).start()
        return 0
    nzfires = (_NCHUNK - pcv) * (_CHUNK // _ZROWS)
    lax.fori_loop(0, nzfires, _zfill, 0)

    mt0 = (w % 2) * _RPW
    mnv = jnp.clip(end_b - start_b - mt0, 0, _RPW)
    for j in range(_RPW // _LANES):
        m = jnp.where(j * _LANES + iota < mnv, 1.0, 0.0).astype(jnp.float32)
        mbuf[pl.ds(j * _LANES, _LANES)] = m
    pltpu.make_async_copy(
        mbuf, mask_hbm.at[pl.ds(pl.multiple_of(w * _RPW, _RPW), _RPW)], msem
    ).start()

    def _pipe(g, _):
        for bi in range(3):
            c = g * 3 + bi

            @pl.when(c < pcv)
            def _(c=c, bi=bi):
                pltpu.make_async_copy(
                    flat_hbm.at[idxs[bi]], bufs[bi], isems[bi]
                ).wait()
                nvc = jnp.clip(nv - c * _CHUNK, 0, _CHUNK)

                def _ztail(row, _c):
                    for kk in range(_D // _LANES):
                        bufs[bi][row, pl.ds(kk * _LANES, _LANES)] = zerosf
                    return 0
                lax.fori_loop(nvc, _CHUNK, _ztail, 0)

                pltpu.make_async_copy(
                    bufs[bi],
                    out_hbm.at[pl.ds(
                        pl.multiple_of(obase + c * _CHUNK, _CHUNK), _CHUNK)],
                    osems[bi],
                ).start()

                @pl.when(c + 3 < pcv)
                def _():
                    pltpu.make_async_copy(
                        bufs[bi],
                        out_hbm.at[pl.ds(0, _CHUNK)],
                        osems[bi],
                    ).wait()
                    start_in(c + 3, bi)
        return 0
    lax.fori_loop(0, (pcv + 2) // 3, _pipe, 0)

    for bi in range(3):
        @pl.when(bi < pcv)
        def _(bi=bi):
            pltpu.make_async_copy(
                bufs[bi],
                out_hbm.at[pl.ds(0, _CHUNK)],
                osems[bi],
            ).wait()

    pltpu.make_async_copy(
        mbuf, mask_hbm.at[pl.ds(0, _RPW)], msem).wait()

    def _zdrain(h, _):
        pltpu.make_async_copy(zbuf, out_hbm.at[pl.ds(0, _ZROWS)], zsem).wait()
        return 0
    lax.fori_loop(0, nzfires, _zdrain, 0)


@jax.jit
def _padded_gather(flat, cu):
    mesh = plsc.VectorSubcoreMesh(core_axis_name="c", subcore_axis_name="s")
    return pl.kernel(
        _sc_body,
        out_type=(
            jax.ShapeDtypeStruct((_B * _L, _D), jnp.float32),
            jax.ShapeDtypeStruct((_B * _L,), jnp.float32),
        ),
        mesh=mesh,
        scratch_types=[
            pltpu.VMEM((3 * _LANES,), jnp.int32),
            pltpu.VMEM((_CHUNK,), jnp.int32),
            pltpu.VMEM((_CHUNK,), jnp.int32),
            pltpu.VMEM((_CHUNK,), jnp.int32),
            pltpu.VMEM((_CHUNK, _D), jnp.float32),
            pltpu.VMEM((_CHUNK, _D), jnp.float32),
            pltpu.VMEM((_CHUNK, _D), jnp.float32),
            pltpu.VMEM((_ZROWS, _D), jnp.float32),
            pltpu.VMEM((_RPW,), jnp.float32),
            pltpu.SemaphoreType.DMA,
            pltpu.SemaphoreType.DMA,
            pltpu.SemaphoreType.DMA,
            pltpu.SemaphoreType.DMA,
            pltpu.SemaphoreType.DMA,
            pltpu.SemaphoreType.DMA,
            pltpu.SemaphoreType.DMA,
            pltpu.SemaphoreType.DMA,
        ],
    )(flat, cu)


def kernel(flat, cu_seqlens):
    return _padded_gather(flat, cu_seqlens)


# speedup vs baseline: 1.0117x; 1.0030x over previous
import jax
import jax.numpy as jnp
from jax import lax
from jax.experimental import pallas as pl
from jax.experimental.pallas import tpu as pltpu
from jax.experimental.pallas import tpu_sc as plsc

_B = 16
_L = 4096
_D = 512
_TOTAL = _B * _L // 2
_NW = 32
_RPW = _B * _L // _NW
_CHUNK = 64
_NCHUNK = _RPW // _CHUNK
_ZROWS = 32
_LANES = 16


def _sc_body(flat_hbm, cu_hbm, out_hbm, mask_hbm,
             cu_v, idx0, idx1, idx2, buf0, buf1, buf2, zbuf, mbuf,
             isem0, isem1, isem2, osem0, osem1, osem2, zsem, msem):
    cid = lax.axis_index("c")
    sid = lax.axis_index("s")
    w = sid * 2 + cid
    b = w // 2
    half = (w % 2) ^ (b % 2)
    t0 = half * _RPW
    obase = (2 * b + half) * _RPW

    pltpu.sync_copy(cu_hbm, cu_v.at[pl.ds(0, _B + 1)])
    iota = lax.iota(jnp.int32, _LANES)
    start_b = cu_v[pl.ds(b, _LANES)][0]
    end_b = cu_v[pl.ds(b + 1, _LANES)][0]
    nv = jnp.clip(end_b - start_b - t0, 0, _RPW)
    s0 = start_b + t0
    pcv = (nv + _CHUNK - 1) // _CHUNK

    bufs = (buf0, buf1, buf2)
    idxs = (idx0, idx1, idx2)
    isems = (isem0, isem1, isem2)
    osems = (osem0, osem1, osem2)

    def start_in(c, bi):
        s = s0 + c * _CHUNK
        for kk in range(_CHUNK // _LANES):
            v = jnp.minimum(s + kk * _LANES + iota, _TOTAL - 1)
            idxs[bi][pl.ds(kk * _LANES, _LANES)] = v
        pltpu.make_async_copy(flat_hbm.at[idxs[bi]], bufs[bi], isems[bi]).start()

    for bi in range(3):
        @pl.when(bi < pcv)
        def _(bi=bi):
            start_in(bi, bi)

    zerosf = jnp.zeros((_LANES,), jnp.float32)

    def _zrow(row, _):
        for kk in range(_D // _LANES):
            zbuf[row, pl.ds(kk * _LANES, _LANES)] = zerosf
        return 0
    lax.fori_loop(0, _ZROWS, _zrow, 0)

    def _zfill(h, _):
        dst = pl.multiple_of(obase + pcv * _CHUNK + h * _ZROWS, _ZROWS)
        pltpu.make_async_copy(zbuf, out_hbm.at[pl.ds(dst, _ZROWS)], zsem).start()
        return 0
    nzfires = (_NCHUNK - pcv) * (_CHUNK // _ZROWS)
    lax.fori_loop(0, nzfires, _zfill, 0)

    mt0 = (w % 2) * _RPW
    mnv = jnp.clip(end_b - start_b - mt0, 0, _RPW)
    for j in range(_RPW // _LANES):
        m = jnp.where(j * _LANES + iota < mnv, 1.0, 0.0).astype(jnp.float32)
        mbuf[pl.ds(j * _LANES, _LANES)] = m
    pltpu.make_async_copy(
        mbuf, mask_hbm.at[pl.ds(pl.multiple_of(w * _RPW, _RPW), _RPW)], msem
    ).start()

    def _pipe(g, _):
        for bi in range(3):
            c = g * 3 + bi

            @pl.when(c < pcv)
            def _(c=c, bi=bi):
                pltpu.make_async_copy(
                    flat_hbm.at[idxs[bi]], bufs[bi], isems[bi]
                ).wait()
                nvc = jnp.clip(nv - c * _CHUNK, 0, _CHUNK)

                def _ztail(row, _c):
                    for kk in range(_D // _LANES):
                        bufs[bi][row, pl.ds(kk * _LANES, _LANES)] = zerosf
                    return 0
                lax.fori_loop(nvc, _CHUNK, _ztail, 0)

                pltpu.make_async_copy(
                    bufs[bi],
                    out_hbm.at[pl.ds(
                        pl.multiple_of(obase + c * _CHUNK, _CHUNK), _CHUNK)],
                    osems[bi],
                ).start()

                @pl.when(c + 3 < pcv)
                def _():
                    pltpu.make_async_copy(
                        bufs[bi],
                        out_hbm.at[pl.ds(0, _CHUNK)],
                        osems[bi],
                    ).wait()
                    start_in(c + 3, bi)
        return 0
    lax.fori_loop(0, (pcv + 2) // 3, _pipe, 0)

    for bi in range(3):
        @pl.when(bi < pcv)
        def _(bi=bi):
            pltpu.make_async_copy(
                bufs[bi],
                out_hbm.at[pl.ds(0, _CHUNK)],
                osems[bi],
            ).wait()

    pltpu.make_async_copy(
        mbuf, mask_hbm.at[pl.ds(0, _RPW)], msem).wait()

    def _zdrain(h, _):
        pltpu.make_async_copy(zbuf, out_hbm.at[pl.ds(0, _ZROWS)], zsem).wait()
        return 0
    lax.fori_loop(0, nzfires, _zdrain, 0)


@jax.jit
def _padded_gather(flat, cu):
    mesh = plsc.VectorSubcoreMesh(core_axis_name="c", subcore_axis_name="s")
    return pl.kernel(
        _sc_body,
        out_type=(
            jax.ShapeDtypeStruct((_B * _L, _D), jnp.float32),
            jax.ShapeDtypeStruct((_B * _L,), jnp.float32),
        ),
        mesh=mesh,
        scratch_types=[
            pltpu.VMEM((3 * _LANES,), jnp.int32),
            pltpu.VMEM((_CHUNK,), jnp.int32),
            pltpu.VMEM((_CHUNK,), jnp.int32),
            pltpu.VMEM((_CHUNK,), jnp.int32),
            pltpu.VMEM((_CHUNK, _D), jnp.float32),
            pltpu.VMEM((_CHUNK, _D), jnp.float32),
            pltpu.VMEM((_CHUNK, _D), jnp.float32),
            pltpu.VMEM((_ZROWS, _D), jnp.float32),
            pltpu.VMEM((_RPW,), jnp.float32),
            pltpu.SemaphoreType.DMA,
            pltpu.SemaphoreType.DMA,
            pltpu.SemaphoreType.DMA,
            pltpu.SemaphoreType.DMA,
            pltpu.SemaphoreType.DMA,
            pltpu.SemaphoreType.DMA,
            pltpu.SemaphoreType.DMA,
            pltpu.SemaphoreType.DMA,
        ],
    )(flat, cu)


def kernel(flat, cu_seqlens):
    return _padded_gather(flat, cu_seqlens)
